# baseline (speedup 1.0000x reference)
import functools

import jax
import jax.numpy as jnp
from jax import lax
from jax.experimental import pallas as pl
from jax.experimental.pallas import tpu as pltpu
from jax.experimental.pallas import tpu_sc as plsc

N = 10000
E = 320000
C = 64
ED = 128
Z = 2 * C + ED

NC = 2
NS = 16
NW = NC * NS
RPT = N // NS

CHAINS = ((0, 38400), (38400, 89600), (128000, 102400), (230400, 89600))

G_CH = 40
G_K = 5
G_WV = G_K * G_CH
G_R = 2

S_CH = 40
S_K = 5
S_WV = S_K * S_CH


def _sc_gather_body(ept, nwave, e_off, x_hbm, ei_hbm, xij_hbm,
                    tbl, dst_v, src_v, bi, bj, sem_g, sem_w):
    cid = lax.axis_index("c")
    sid = lax.axis_index("s")
    wid = sid * NC + cid
    base = wid * ept
    rows = sid * RPT
    pltpu.sync_copy(x_hbm.at[pl.ds(rows, RPT)], tbl.at[pl.ds(rows, RPT)])
    pltpu.sync_copy(ei_hbm.at[1, pl.ds(e_off + base, ept)], dst_v)
    pltpu.sync_copy(ei_hbm.at[0, pl.ds(e_off + base, ept)], src_v)
    plsc.subcore_barrier()

    def wave(w, carry):
        s = w % G_R

        @pl.when(w >= G_R)
        def _drain():
            pltpu.make_async_copy(
                bi.at[s], xij_hbm.at[pl.ds(base, G_WV), pl.ds(0, C)], sem_w).wait()
            pltpu.make_async_copy(
                bj.at[s], xij_hbm.at[pl.ds(base, G_WV), pl.ds(C, C)], sem_w).wait()

        cps = []
        for b in range(G_K):
            o = w * G_WV + b * G_CH
            cps.append(pltpu.async_copy(
                tbl.at[dst_v.at[pl.ds(o, G_CH)]],
                bi.at[s, pl.ds(b * G_CH, G_CH)], sem_g))
            cps.append(pltpu.async_copy(
                tbl.at[src_v.at[pl.ds(o, G_CH)]],
                bj.at[s, pl.ds(b * G_CH, G_CH)], sem_g))
        for cp in cps:
            cp.wait()
        o = base + w * G_WV
        pltpu.async_copy(bi.at[s], xij_hbm.at[pl.ds(o, G_WV), pl.ds(0, C)], sem_w)
        pltpu.async_copy(bj.at[s], xij_hbm.at[pl.ds(o, G_WV), pl.ds(C, C)], sem_w)
        return carry

    lax.fori_loop(0, nwave, wave, 0)
    for _ in range(G_R):
        pltpu.make_async_copy(
            bi.at[0], xij_hbm.at[pl.ds(base, G_WV), pl.ds(0, C)], sem_w).wait()
        pltpu.make_async_copy(
            bj.at[0], xij_hbm.at[pl.ds(base, G_WV), pl.ds(C, C)], sem_w).wait()


def _sc_scatter_body(rpt, nwave, e_off, eh, msg_hbm, ei_hbm, zero_hbm, out_hbm,
                     mba, mbb, iba, ibb, acc, sem_l, sem_s):
    cid = lax.axis_index("c")
    sid = lax.axis_index("s")
    wid = sid * NC + cid
    base = wid * rpt
    rows = sid * RPT
    pltpu.sync_copy(zero_hbm.at[pl.ds(rows, RPT)], acc.at[pl.ds(rows, RPT)])
    plsc.subcore_barrier()

    def wave(w, carry):
        loads = []
        for b in range(S_K):
            o = base + w * S_WV + b * S_CH
            loads.append(pltpu.async_copy(
                ei_hbm.at[1, pl.ds(e_off + o, S_CH)], iba.at[b], sem_l))
            loads.append(pltpu.async_copy(
                ei_hbm.at[1, pl.ds(e_off + eh + o, S_CH)], ibb.at[b], sem_l))
            loads.append(pltpu.async_copy(
                msg_hbm.at[pl.ds(o, S_CH), pl.ds(0, C)], mba.at[b], sem_l))
            loads.append(pltpu.async_copy(
                msg_hbm.at[pl.ds(o, S_CH), pl.ds(C, C)], mbb.at[b], sem_l))
        for cp in loads:
            cp.wait()
        adds = []
        for b in range(S_K):
            adds.append(pltpu.async_copy(
                mba.at[b], acc.at[iba.at[b]], sem_s, add=True))
            adds.append(pltpu.async_copy(
                mbb.at[b], acc.at[ibb.at[b]], sem_s, add=True))
        for cp in adds:
            cp.wait()
        return carry

    lax.fori_loop(0, nwave, wave, 0)
    plsc.subcore_barrier()
    pltpu.sync_copy(acc.at[pl.ds(rows, RPT)],
                    out_hbm.at[pl.ds(rows, RPT), pl.ds(cid * C, C)])


@functools.cache
def _sc_kernels():
    mesh = plsc.VectorSubcoreMesh(core_axis_name="c", subcore_axis_name="s",
                                  num_cores=NC, num_subcores=NS)
    params = pltpu.CompilerParams(use_tc_tiling_on_sc=False)
    gathers, scatters = [], []
    for e_off, ec in CHAINS:
        ept = ec // NW
        gathers.append(pl.kernel(
            functools.partial(_sc_gather_body, ept, ept // G_WV, e_off),
            out_type=jax.ShapeDtypeStruct((ec, 2 * C), jnp.float32),
            mesh=mesh,
            compiler_params=params,
            scratch_types=[
                pltpu.VMEM_SHARED((N, C), jnp.float32),
                pltpu.VMEM((ept,), jnp.int32),
                pltpu.VMEM((ept,), jnp.int32),
                pltpu.VMEM((G_R, G_WV, C), jnp.float32),
                pltpu.VMEM((G_R, G_WV, C), jnp.float32),
                pltpu.SemaphoreType.DMA,
                pltpu.SemaphoreType.DMA,
            ],
        ))
        rpt = (ec // 2) // NW
        scatters.append(pl.kernel(
            functools.partial(_sc_scatter_body, rpt, rpt // S_WV, e_off, ec // 2),
            out_type=jax.ShapeDtypeStruct((N, 2 * C), jnp.float32),
            mesh=mesh,
            compiler_params=params,
            scratch_types=[
                pltpu.VMEM((S_K, S_CH, C), jnp.float32),
                pltpu.VMEM((S_K, S_CH, C), jnp.float32),
                pltpu.VMEM((S_K, S_CH), jnp.int32),
                pltpu.VMEM((S_K, S_CH), jnp.int32),
                pltpu.VMEM_SHARED((N, C), jnp.float32),
                pltpu.SemaphoreType.DMA,
                pltpu.SemaphoreType.DMA,
            ],
        ))
    return gathers, scatters


BH = 3200


def _dense_body(xa_ref, xb_ref, ea_ref, eb_ref, w_ref, b_ref, out_ref):
    za = jnp.concatenate([xa_ref[...], ea_ref[...]], axis=-1)
    zb = jnp.concatenate([xb_ref[...], eb_ref[...]], axis=-1)
    ga = jnp.dot(za, w_ref[...], preferred_element_type=jnp.float32) + b_ref[...]
    gb = jnp.dot(zb, w_ref[...], preferred_element_type=jnp.float32) + b_ref[...]

    def act(gs):
        g = gs[:, :C]
        s = gs[:, C:]
        gate = 0.5 + 0.5 * jnp.tanh(0.5 * g)
        core = jnp.maximum(s, 0.0) + jnp.log1p(jnp.exp(-jnp.abs(s)))
        return gate * core

    out_ref[...] = jnp.concatenate([act(ga), act(gb)], axis=-1)


def _dense(xij, edge_attr, w_cat, b_cat, e_off, ec):
    eh = ec // 2
    nblk = eh // BH
    ea_a = e_off // BH
    ea_b = (e_off + eh) // BH
    return pl.pallas_call(
        _dense_body,
        grid=(nblk,),
        in_specs=[
            pl.BlockSpec((BH, 2 * C), lambda i: (i, 0)),
            pl.BlockSpec((BH, 2 * C), lambda i, n=nblk: (i + n, 0)),
            pl.BlockSpec((BH, ED), lambda i, o=ea_a: (i + o, 0)),
            pl.BlockSpec((BH, ED), lambda i, o=ea_b: (i + o, 0)),
            pl.BlockSpec((Z, 2 * C), lambda i: (0, 0)),
            pl.BlockSpec((1, 2 * C), lambda i: (0, 0)),
        ],
        out_specs=pl.BlockSpec((BH, 2 * C), lambda i: (i, 0)),
        out_shape=jax.ShapeDtypeStruct((eh, 2 * C), jnp.float32),
    )(xij, xij, edge_attr, edge_attr, w_cat, b_cat)


BN = 2000


def _combine_body(x_ref, *refs):
    p_refs, out_ref = refs[:-1], refs[-1]
    s = x_ref[...]
    for p in p_refs:
        s = s + p[:, :C] + p[:, C:]
    out_ref[...] = jnp.maximum(s, 0.0)


def _combine(x, partials):
    return pl.pallas_call(
        _combine_body,
        grid=(N // BN,),
        in_specs=[pl.BlockSpec((BN, C), lambda i: (i, 0))]
        + [pl.BlockSpec((BN, 2 * C), lambda i: (i, 0)) for _ in partials],
        out_specs=pl.BlockSpec((BN, C), lambda i: (i, 0)),
        out_shape=jax.ShapeDtypeStruct((N, C), jnp.float32),
    )(x, *partials)


def kernel(x, edge_index, edge_attr, W_f, b_f, W_s, b_s):
    gathers, scatters = _sc_kernels()
    ei = edge_index if edge_index.dtype == jnp.int32 else edge_index.astype(jnp.int32)
    w_cat = jnp.concatenate([W_f, W_s], axis=1)
    b_cat = jnp.concatenate([b_f, b_s]).reshape(1, 2 * C)
    zeros = jnp.zeros((N, C), jnp.float32)
    partials = []
    for (e_off, ec), g, s in zip(CHAINS, gathers, scatters):
        xij = g(x, ei)
        msg2 = _dense(xij, edge_attr, w_cat, b_cat, e_off, ec)
        partials.append(s(msg2, ei, zeros))
    return _combine(x, partials)


# speedup vs baseline: 1.0757x; 1.0757x over previous
import functools

import jax
import jax.numpy as jnp
from jax import lax
from jax.experimental import pallas as pl
from jax.experimental.pallas import tpu as pltpu
from jax.experimental.pallas import tpu_sc as plsc

N = 10000
E = 320000
C = 64
ED = 128
Z = 2 * C + ED

NC = 2
NS = 16
NW = NC * NS
RPT = N // NS

CHAINS = ((0, 115200), (115200, 115200), (230400, 89600))

G_CH = 40
G_K = 5
G_WV = G_K * G_CH
G_R = 2

S_CH = 40
S_K = 5
S_WV = S_K * S_CH


def _sc_gather_body(ept, nwave, e_off, x_hbm, ei_hbm, xij_hbm,
                    tbl, dst_v, src_v, bi, bj, sem_g, sem_w):
    cid = lax.axis_index("c")
    sid = lax.axis_index("s")
    wid = sid * NC + cid
    base = wid * ept
    rows = sid * RPT
    pltpu.sync_copy(x_hbm.at[pl.ds(rows, RPT)], tbl.at[pl.ds(rows, RPT)])
    pltpu.sync_copy(ei_hbm.at[1, pl.ds(e_off + base, ept)], dst_v)
    pltpu.sync_copy(ei_hbm.at[0, pl.ds(e_off + base, ept)], src_v)
    plsc.subcore_barrier()

    def wave(w, carry):
        s = w % G_R

        @pl.when(w >= G_R)
        def _drain():
            pltpu.make_async_copy(
                bi.at[s], xij_hbm.at[pl.ds(base, G_WV), pl.ds(0, C)], sem_w).wait()
            pltpu.make_async_copy(
                bj.at[s], xij_hbm.at[pl.ds(base, G_WV), pl.ds(C, C)], sem_w).wait()

        cps = []
        for b in range(G_K):
            o = w * G_WV + b * G_CH
            cps.append(pltpu.async_copy(
                tbl.at[dst_v.at[pl.ds(o, G_CH)]],
                bi.at[s, pl.ds(b * G_CH, G_CH)], sem_g))
            cps.append(pltpu.async_copy(
                tbl.at[src_v.at[pl.ds(o, G_CH)]],
                bj.at[s, pl.ds(b * G_CH, G_CH)], sem_g))
        for cp in cps:
            cp.wait()
        o = base + w * G_WV
        pltpu.async_copy(bi.at[s], xij_hbm.at[pl.ds(o, G_WV), pl.ds(0, C)], sem_w)
        pltpu.async_copy(bj.at[s], xij_hbm.at[pl.ds(o, G_WV), pl.ds(C, C)], sem_w)
        return carry

    lax.fori_loop(0, nwave, wave, 0)
    for _ in range(G_R):
        pltpu.make_async_copy(
            bi.at[0], xij_hbm.at[pl.ds(base, G_WV), pl.ds(0, C)], sem_w).wait()
        pltpu.make_async_copy(
            bj.at[0], xij_hbm.at[pl.ds(base, G_WV), pl.ds(C, C)], sem_w).wait()


def _sc_scatter_body(rpt, nwave, e_off, eh, msg_hbm, ei_hbm, zero_hbm, out_hbm,
                     mba, mbb, iba, ibb, acc, sem_l, sem_s):
    cid = lax.axis_index("c")
    sid = lax.axis_index("s")
    wid = sid * NC + cid
    base = wid * rpt
    rows = sid * RPT
    pltpu.sync_copy(zero_hbm.at[pl.ds(rows, RPT)], acc.at[pl.ds(rows, RPT)])
    plsc.subcore_barrier()

    def wave(w, carry):
        loads = []
        for b in range(S_K):
            o = base + w * S_WV + b * S_CH
            loads.append(pltpu.async_copy(
                ei_hbm.at[1, pl.ds(e_off + o, S_CH)], iba.at[b], sem_l))
            loads.append(pltpu.async_copy(
                ei_hbm.at[1, pl.ds(e_off + eh + o, S_CH)], ibb.at[b], sem_l))
            loads.append(pltpu.async_copy(
                msg_hbm.at[pl.ds(o, S_CH), pl.ds(0, C)], mba.at[b], sem_l))
            loads.append(pltpu.async_copy(
                msg_hbm.at[pl.ds(o, S_CH), pl.ds(C, C)], mbb.at[b], sem_l))
        for cp in loads:
            cp.wait()
        adds = []
        for b in range(S_K):
            adds.append(pltpu.async_copy(
                mba.at[b], acc.at[iba.at[b]], sem_s, add=True))
            adds.append(pltpu.async_copy(
                mbb.at[b], acc.at[ibb.at[b]], sem_s, add=True))
        for cp in adds:
            cp.wait()
        return carry

    lax.fori_loop(0, nwave, wave, 0)
    plsc.subcore_barrier()
    pltpu.sync_copy(acc.at[pl.ds(rows, RPT)],
                    out_hbm.at[pl.ds(rows, RPT), pl.ds(cid * C, C)])


@functools.cache
def _sc_kernels():
    mesh = plsc.VectorSubcoreMesh(core_axis_name="c", subcore_axis_name="s",
                                  num_cores=NC, num_subcores=NS)
    params = pltpu.CompilerParams(use_tc_tiling_on_sc=False)
    gathers, scatters = [], []
    for e_off, ec in CHAINS:
        ept = ec // NW
        gathers.append(pl.kernel(
            functools.partial(_sc_gather_body, ept, ept // G_WV, e_off),
            out_type=jax.ShapeDtypeStruct((ec, 2 * C), jnp.float32),
            mesh=mesh,
            compiler_params=params,
            scratch_types=[
                pltpu.VMEM_SHARED((N, C), jnp.float32),
                pltpu.VMEM((ept,), jnp.int32),
                pltpu.VMEM((ept,), jnp.int32),
                pltpu.VMEM((G_R, G_WV, C), jnp.float32),
                pltpu.VMEM((G_R, G_WV, C), jnp.float32),
                pltpu.SemaphoreType.DMA,
                pltpu.SemaphoreType.DMA,
            ],
        ))
        rpt = (ec // 2) // NW
        scatters.append(pl.kernel(
            functools.partial(_sc_scatter_body, rpt, rpt // S_WV, e_off, ec // 2),
            out_type=jax.ShapeDtypeStruct((N, 2 * C), jnp.float32),
            mesh=mesh,
            compiler_params=params,
            scratch_types=[
                pltpu.VMEM((S_K, S_CH, C), jnp.float32),
                pltpu.VMEM((S_K, S_CH, C), jnp.float32),
                pltpu.VMEM((S_K, S_CH), jnp.int32),
                pltpu.VMEM((S_K, S_CH), jnp.int32),
                pltpu.VMEM_SHARED((N, C), jnp.float32),
                pltpu.SemaphoreType.DMA,
                pltpu.SemaphoreType.DMA,
            ],
        ))
    return gathers, scatters


BH = 3200


def _dense_body(xa_ref, xb_ref, ea_ref, eb_ref, w_ref, b_ref, out_ref):
    za = jnp.concatenate([xa_ref[...], ea_ref[...]], axis=-1)
    zb = jnp.concatenate([xb_ref[...], eb_ref[...]], axis=-1)
    ga = jnp.dot(za, w_ref[...], preferred_element_type=jnp.float32) + b_ref[...]
    gb = jnp.dot(zb, w_ref[...], preferred_element_type=jnp.float32) + b_ref[...]

    def act(gs):
        g = gs[:, :C]
        s = gs[:, C:]
        gate = 0.5 + 0.5 * jnp.tanh(0.5 * g)
        core = jnp.maximum(s, 0.0) + jnp.log1p(jnp.exp(-jnp.abs(s)))
        return gate * core

    out_ref[...] = jnp.concatenate([act(ga), act(gb)], axis=-1)


def _dense(xij, edge_attr, w_cat, b_cat, e_off, ec):
    eh = ec // 2
    nblk = eh // BH
    ea_a = e_off // BH
    ea_b = (e_off + eh) // BH
    return pl.pallas_call(
        _dense_body,
        grid=(nblk,),
        in_specs=[
            pl.BlockSpec((BH, 2 * C), lambda i: (i, 0)),
            pl.BlockSpec((BH, 2 * C), lambda i, n=nblk: (i + n, 0)),
            pl.BlockSpec((BH, ED), lambda i, o=ea_a: (i + o, 0)),
            pl.BlockSpec((BH, ED), lambda i, o=ea_b: (i + o, 0)),
            pl.BlockSpec((Z, 2 * C), lambda i: (0, 0)),
            pl.BlockSpec((1, 2 * C), lambda i: (0, 0)),
        ],
        out_specs=pl.BlockSpec((BH, 2 * C), lambda i: (i, 0)),
        out_shape=jax.ShapeDtypeStruct((eh, 2 * C), jnp.float32),
    )(xij, xij, edge_attr, edge_attr, w_cat, b_cat)


BN = 2000


def _combine_body(x_ref, *refs):
    p_refs, out_ref = refs[:-1], refs[-1]
    s = x_ref[...]
    for p in p_refs:
        s = s + p[:, :C] + p[:, C:]
    out_ref[...] = jnp.maximum(s, 0.0)


def _combine(x, partials):
    return pl.pallas_call(
        _combine_body,
        grid=(N // BN,),
        in_specs=[pl.BlockSpec((BN, C), lambda i: (i, 0))]
        + [pl.BlockSpec((BN, 2 * C), lambda i: (i, 0)) for _ in partials],
        out_specs=pl.BlockSpec((BN, C), lambda i: (i, 0)),
        out_shape=jax.ShapeDtypeStruct((N, C), jnp.float32),
    )(x, *partials)


def kernel(x, edge_index, edge_attr, W_f, b_f, W_s, b_s):
    gathers, scatters = _sc_kernels()
    ei = edge_index if edge_index.dtype == jnp.int32 else edge_index.astype(jnp.int32)
    w_cat = jnp.concatenate([W_f, W_s], axis=1)
    b_cat = jnp.concatenate([b_f, b_s]).reshape(1, 2 * C)
    zeros = jnp.zeros((N, C), jnp.float32)
    partials = []
    for (e_off, ec), g, s in zip(CHAINS, gathers, scatters):
        xij = g(x, ei)
        msg2 = _dense(xij, edge_attr, w_cat, b_cat, e_off, ec)
        partials.append(s(msg2, ei, zeros))
    return _combine(x, partials)


# speedup vs baseline: 1.0898x; 1.0131x over previous
import functools

import jax
import jax.numpy as jnp
from jax import lax
from jax.experimental import pallas as pl
from jax.experimental.pallas import tpu as pltpu
from jax.experimental.pallas import tpu_sc as plsc

N = 10000
E = 320000
C = 64
ED = 128
Z = 2 * C + ED

NC = 2
NS = 16
NW = NC * NS
RPT = N // NS

CHAINS = ((0, 89600), (89600, 153600), (243200, 76800))

G_CH = 40
G_K = 5
G_WV = G_K * G_CH
G_R = 2

S_CH = 40
S_K = 5
S_WV = S_K * S_CH


def _sc_gather_body(ept, nwave, e_off, x_hbm, ei_hbm, xij_hbm,
                    tbl, dst_v, src_v, bi, bj, sem_g, sem_w):
    cid = lax.axis_index("c")
    sid = lax.axis_index("s")
    wid = sid * NC + cid
    base = wid * ept
    rows = sid * RPT
    pltpu.sync_copy(x_hbm.at[pl.ds(rows, RPT)], tbl.at[pl.ds(rows, RPT)])
    pltpu.sync_copy(ei_hbm.at[1, pl.ds(e_off + base, ept)], dst_v)
    pltpu.sync_copy(ei_hbm.at[0, pl.ds(e_off + base, ept)], src_v)
    plsc.subcore_barrier()

    def wave(w, carry):
        s = w % G_R

        @pl.when(w >= G_R)
        def _drain():
            pltpu.make_async_copy(
                bi.at[s], xij_hbm.at[pl.ds(base, G_WV), pl.ds(0, C)], sem_w).wait()
            pltpu.make_async_copy(
                bj.at[s], xij_hbm.at[pl.ds(base, G_WV), pl.ds(C, C)], sem_w).wait()

        cps = []
        for b in range(G_K):
            o = w * G_WV + b * G_CH
            cps.append(pltpu.async_copy(
                tbl.at[dst_v.at[pl.ds(o, G_CH)]],
                bi.at[s, pl.ds(b * G_CH, G_CH)], sem_g))
            cps.append(pltpu.async_copy(
                tbl.at[src_v.at[pl.ds(o, G_CH)]],
                bj.at[s, pl.ds(b * G_CH, G_CH)], sem_g))
        for cp in cps:
            cp.wait()
        o = base + w * G_WV
        pltpu.async_copy(bi.at[s], xij_hbm.at[pl.ds(o, G_WV), pl.ds(0, C)], sem_w)
        pltpu.async_copy(bj.at[s], xij_hbm.at[pl.ds(o, G_WV), pl.ds(C, C)], sem_w)
        return carry

    lax.fori_loop(0, nwave, wave, 0)
    for _ in range(G_R):
        pltpu.make_async_copy(
            bi.at[0], xij_hbm.at[pl.ds(base, G_WV), pl.ds(0, C)], sem_w).wait()
        pltpu.make_async_copy(
            bj.at[0], xij_hbm.at[pl.ds(base, G_WV), pl.ds(C, C)], sem_w).wait()


def _sc_scatter_body(rpt, nwave, e_off, eh, msg_hbm, ei_hbm, prev_hbm, out_hbm,
                     mba, mbb, iba, ibb, acc, sem_l, sem_s):
    cid = lax.axis_index("c")
    sid = lax.axis_index("s")
    wid = sid * NC + cid
    base = wid * rpt
    rows = sid * RPT
    pltpu.sync_copy(prev_hbm.at[pl.ds(rows, RPT), pl.ds(cid * C, C)],
                    acc.at[pl.ds(rows, RPT)])
    plsc.subcore_barrier()

    def wave(w, carry):
        loads = []
        for b in range(S_K):
            o = base + w * S_WV + b * S_CH
            loads.append(pltpu.async_copy(
                ei_hbm.at[1, pl.ds(e_off + o, S_CH)], iba.at[b], sem_l))
            loads.append(pltpu.async_copy(
                ei_hbm.at[1, pl.ds(e_off + eh + o, S_CH)], ibb.at[b], sem_l))
            loads.append(pltpu.async_copy(
                msg_hbm.at[pl.ds(o, S_CH), pl.ds(0, C)], mba.at[b], sem_l))
            loads.append(pltpu.async_copy(
                msg_hbm.at[pl.ds(o, S_CH), pl.ds(C, C)], mbb.at[b], sem_l))
        for cp in loads:
            cp.wait()
        adds = []
        for b in range(S_K):
            adds.append(pltpu.async_copy(
                mba.at[b], acc.at[iba.at[b]], sem_s, add=True))
            adds.append(pltpu.async_copy(
                mbb.at[b], acc.at[ibb.at[b]], sem_s, add=True))
        for cp in adds:
            cp.wait()
        return carry

    lax.fori_loop(0, nwave, wave, 0)
    plsc.subcore_barrier()
    pltpu.sync_copy(acc.at[pl.ds(rows, RPT)],
                    out_hbm.at[pl.ds(rows, RPT), pl.ds(cid * C, C)])


@functools.cache
def _sc_kernels():
    mesh = plsc.VectorSubcoreMesh(core_axis_name="c", subcore_axis_name="s",
                                  num_cores=NC, num_subcores=NS)
    params = pltpu.CompilerParams(use_tc_tiling_on_sc=False)
    gathers, scatters = [], []
    for e_off, ec in CHAINS:
        ept = ec // NW
        gathers.append(pl.kernel(
            functools.partial(_sc_gather_body, ept, ept // G_WV, e_off),
            out_type=jax.ShapeDtypeStruct((ec, 2 * C), jnp.float32),
            mesh=mesh,
            compiler_params=params,
            scratch_types=[
                pltpu.VMEM_SHARED((N, C), jnp.float32),
                pltpu.VMEM((ept,), jnp.int32),
                pltpu.VMEM((ept,), jnp.int32),
                pltpu.VMEM((G_R, G_WV, C), jnp.float32),
                pltpu.VMEM((G_R, G_WV, C), jnp.float32),
                pltpu.SemaphoreType.DMA,
                pltpu.SemaphoreType.DMA,
            ],
        ))
        rpt = (ec // 2) // NW
        scatters.append(pl.kernel(
            functools.partial(_sc_scatter_body, rpt, rpt // S_WV, e_off, ec // 2),
            out_type=jax.ShapeDtypeStruct((N, 2 * C), jnp.float32),
            mesh=mesh,
            compiler_params=params,
            scratch_types=[
                pltpu.VMEM((S_K, S_CH, C), jnp.float32),
                pltpu.VMEM((S_K, S_CH, C), jnp.float32),
                pltpu.VMEM((S_K, S_CH), jnp.int32),
                pltpu.VMEM((S_K, S_CH), jnp.int32),
                pltpu.VMEM_SHARED((N, C), jnp.float32),
                pltpu.SemaphoreType.DMA,
                pltpu.SemaphoreType.DMA,
            ],
        ))
    return gathers, scatters


BH = 3200


def _dense_body(xa_ref, xb_ref, ea_ref, eb_ref, w_ref, b_ref, out_ref):
    za = jnp.concatenate([xa_ref[...], ea_ref[...]], axis=-1)
    zb = jnp.concatenate([xb_ref[...], eb_ref[...]], axis=-1)
    ga = jnp.dot(za, w_ref[...], preferred_element_type=jnp.float32) + b_ref[...]
    gb = jnp.dot(zb, w_ref[...], preferred_element_type=jnp.float32) + b_ref[...]

    def act(gs):
        g = gs[:, :C]
        s = gs[:, C:]
        gate = 0.5 + 0.5 * jnp.tanh(0.5 * g)
        core = jnp.maximum(s, 0.0) + jnp.log1p(jnp.exp(-jnp.abs(s)))
        return gate * core

    out_ref[...] = jnp.concatenate([act(ga), act(gb)], axis=-1)


def _dense(xij, edge_attr, w_cat, b_cat, e_off, ec):
    eh = ec // 2
    nblk = eh // BH
    ea_a = e_off // BH
    ea_b = (e_off + eh) // BH
    return pl.pallas_call(
        _dense_body,
        grid=(nblk,),
        in_specs=[
            pl.BlockSpec((BH, 2 * C), lambda i: (i, 0)),
            pl.BlockSpec((BH, 2 * C), lambda i, n=nblk: (i + n, 0)),
            pl.BlockSpec((BH, ED), lambda i, o=ea_a: (i + o, 0)),
            pl.BlockSpec((BH, ED), lambda i, o=ea_b: (i + o, 0)),
            pl.BlockSpec((Z, 2 * C), lambda i: (0, 0)),
            pl.BlockSpec((1, 2 * C), lambda i: (0, 0)),
        ],
        out_specs=pl.BlockSpec((BH, 2 * C), lambda i: (i, 0)),
        out_shape=jax.ShapeDtypeStruct((eh, 2 * C), jnp.float32),
    )(xij, xij, edge_attr, edge_attr, w_cat, b_cat)


BN = 2000


def _combine_body(x_ref, *refs):
    p_refs, out_ref = refs[:-1], refs[-1]
    s = x_ref[...]
    for p in p_refs:
        s = s + p[:, :C] + p[:, C:]
    out_ref[...] = jnp.maximum(s, 0.0)


def _combine(x, partials):
    return pl.pallas_call(
        _combine_body,
        grid=(N // BN,),
        in_specs=[pl.BlockSpec((BN, C), lambda i: (i, 0))]
        + [pl.BlockSpec((BN, 2 * C), lambda i: (i, 0)) for _ in partials],
        out_specs=pl.BlockSpec((BN, C), lambda i: (i, 0)),
        out_shape=jax.ShapeDtypeStruct((N, C), jnp.float32),
    )(x, *partials)


def kernel(x, edge_index, edge_attr, W_f, b_f, W_s, b_s):
    gathers, scatters = _sc_kernels()
    ei = edge_index if edge_index.dtype == jnp.int32 else edge_index.astype(jnp.int32)
    w_cat = jnp.concatenate([W_f, W_s], axis=1)
    b_cat = jnp.concatenate([b_f, b_s]).reshape(1, 2 * C)
    prev = jnp.zeros((N, 2 * C), jnp.float32)
    for (e_off, ec), g, s in zip(CHAINS, gathers, scatters):
        xij = g(x, ei)
        msg2 = _dense(xij, edge_attr, w_cat, b_cat, e_off, ec)
        prev = s(msg2, ei, prev)
    return _combine(x, [prev])


# speedup vs baseline: 1.0986x; 1.0081x over previous
import functools

import jax
import jax.numpy as jnp
from jax import lax
from jax.experimental import pallas as pl
from jax.experimental.pallas import tpu as pltpu
from jax.experimental.pallas import tpu_sc as plsc

N = 10000
E = 320000
C = 64
ED = 128
Z = 2 * C + ED

NC = 2
NS = 16
NW = NC * NS
RPT = N // NS

CHAINS = ((0, 89600), (89600, 153600), (243200, 76800))

G_CH = 40
G_K = 5
G_WV = G_K * G_CH
G_R = 2

S_CH = 40
S_K = 5
S_WV = S_K * S_CH


def _sc_gather_body(ept, nwave, e_off, x_hbm, ei_hbm, xij_hbm,
                    tbl, dst_v, src_v, bi, bj, sem_g, sem_w):
    cid = lax.axis_index("c")
    sid = lax.axis_index("s")
    wid = sid * NC + cid
    base = wid * ept
    rows = sid * RPT
    pltpu.sync_copy(x_hbm.at[pl.ds(rows, RPT)], tbl.at[pl.ds(rows, RPT)])
    pltpu.sync_copy(ei_hbm.at[1, pl.ds(e_off + base, ept)], dst_v)
    pltpu.sync_copy(ei_hbm.at[0, pl.ds(e_off + base, ept)], src_v)
    plsc.subcore_barrier()

    def wave(w, carry):
        s = w % G_R

        @pl.when(w >= G_R)
        def _drain():
            pltpu.make_async_copy(
                bi.at[s], xij_hbm.at[pl.ds(base, G_WV), pl.ds(0, C)], sem_w).wait()
            pltpu.make_async_copy(
                bj.at[s], xij_hbm.at[pl.ds(base, G_WV), pl.ds(C, C)], sem_w).wait()

        cps = []
        for b in range(G_K):
            o = w * G_WV + b * G_CH
            cps.append(pltpu.async_copy(
                tbl.at[dst_v.at[pl.ds(o, G_CH)]],
                bi.at[s, pl.ds(b * G_CH, G_CH)], sem_g))
            cps.append(pltpu.async_copy(
                tbl.at[src_v.at[pl.ds(o, G_CH)]],
                bj.at[s, pl.ds(b * G_CH, G_CH)], sem_g))
        for cp in cps:
            cp.wait()
        o = base + w * G_WV
        pltpu.async_copy(bi.at[s], xij_hbm.at[pl.ds(o, G_WV), pl.ds(0, C)], sem_w)
        pltpu.async_copy(bj.at[s], xij_hbm.at[pl.ds(o, G_WV), pl.ds(C, C)], sem_w)
        return carry

    lax.fori_loop(0, nwave, wave, 0)
    for _ in range(G_R):
        pltpu.make_async_copy(
            bi.at[0], xij_hbm.at[pl.ds(base, G_WV), pl.ds(0, C)], sem_w).wait()
        pltpu.make_async_copy(
            bj.at[0], xij_hbm.at[pl.ds(base, G_WV), pl.ds(C, C)], sem_w).wait()


def _sc_scatter_body(rpt, nwave, e_off, eh, msg_hbm, ei_hbm, prev_hbm, out_hbm,
                     mba, mbb, iba, ibb, acc, sem_l, sem_s):
    cid = lax.axis_index("c")
    sid = lax.axis_index("s")
    wid = sid * NC + cid
    base = wid * rpt
    rows = sid * RPT
    pltpu.sync_copy(prev_hbm.at[pl.ds(rows, RPT), pl.ds(cid * C, C)],
                    acc.at[pl.ds(rows, RPT)])
    plsc.subcore_barrier()

    def wave(w, carry):
        loads = []
        for b in range(S_K):
            o = base + w * S_WV + b * S_CH
            loads.append(pltpu.async_copy(
                ei_hbm.at[1, pl.ds(e_off + o, S_CH)], iba.at[b], sem_l))
            loads.append(pltpu.async_copy(
                ei_hbm.at[1, pl.ds(e_off + eh + o, S_CH)], ibb.at[b], sem_l))
            loads.append(pltpu.async_copy(
                msg_hbm.at[pl.ds(o, S_CH), pl.ds(0, C)], mba.at[b], sem_l))
            loads.append(pltpu.async_copy(
                msg_hbm.at[pl.ds(o, S_CH), pl.ds(C, C)], mbb.at[b], sem_l))
        for cp in loads:
            cp.wait()
        adds = []
        for b in range(S_K):
            adds.append(pltpu.async_copy(
                mba.at[b], acc.at[iba.at[b]], sem_s, add=True))
            adds.append(pltpu.async_copy(
                mbb.at[b], acc.at[ibb.at[b]], sem_s, add=True))
        for cp in adds:
            cp.wait()
        return carry

    lax.fori_loop(0, nwave, wave, 0)
    plsc.subcore_barrier()
    pltpu.sync_copy(acc.at[pl.ds(rows, RPT)],
                    out_hbm.at[pl.ds(rows, RPT), pl.ds(cid * C, C)])


@functools.cache
def _sc_kernels():
    mesh = plsc.VectorSubcoreMesh(core_axis_name="c", subcore_axis_name="s",
                                  num_cores=NC, num_subcores=NS)
    params = pltpu.CompilerParams(use_tc_tiling_on_sc=False)
    gathers, scatters = [], []
    for e_off, ec in CHAINS:
        ept = ec // NW
        gathers.append(pl.kernel(
            functools.partial(_sc_gather_body, ept, ept // G_WV, e_off),
            out_type=jax.ShapeDtypeStruct((ec, 2 * C), jnp.float32),
            mesh=mesh,
            compiler_params=params,
            scratch_types=[
                pltpu.VMEM_SHARED((N, C), jnp.float32),
                pltpu.VMEM((ept,), jnp.int32),
                pltpu.VMEM((ept,), jnp.int32),
                pltpu.VMEM((G_R, G_WV, C), jnp.float32),
                pltpu.VMEM((G_R, G_WV, C), jnp.float32),
                pltpu.SemaphoreType.DMA,
                pltpu.SemaphoreType.DMA,
            ],
        ))
        rpt = (ec // 2) // NW
        scatters.append(pl.kernel(
            functools.partial(_sc_scatter_body, rpt, rpt // S_WV, e_off, ec // 2),
            out_type=jax.ShapeDtypeStruct((N, 2 * C), jnp.float32),
            mesh=mesh,
            compiler_params=params,
            scratch_types=[
                pltpu.VMEM((S_K, S_CH, C), jnp.float32),
                pltpu.VMEM((S_K, S_CH, C), jnp.float32),
                pltpu.VMEM((S_K, S_CH), jnp.int32),
                pltpu.VMEM((S_K, S_CH), jnp.int32),
                pltpu.VMEM_SHARED((N, C), jnp.float32),
                pltpu.SemaphoreType.DMA,
                pltpu.SemaphoreType.DMA,
            ],
        ))
    return gathers, scatters


BH = 3200


def _dense_body(xa_ref, xb_ref, ea_ref, eb_ref, w_ref, b_ref, out_ref):
    za = jnp.concatenate([xa_ref[...], ea_ref[...]], axis=-1)
    zb = jnp.concatenate([xb_ref[...], eb_ref[...]], axis=-1)
    ga = jnp.dot(za, w_ref[...], preferred_element_type=jnp.float32) + b_ref[...]
    gb = jnp.dot(zb, w_ref[...], preferred_element_type=jnp.float32) + b_ref[...]

    def act(gs):
        h = gs.astype(jnp.bfloat16)
        g = h[:, :C]
        s = h[:, C:]
        gate = jnp.bfloat16(0.5) + jnp.bfloat16(0.5) * jnp.tanh(jnp.bfloat16(0.5) * g)
        core = (jnp.maximum(s, jnp.bfloat16(0.0))
                + jnp.log1p(jnp.exp(-jnp.abs(s))))
        return (gate * core).astype(jnp.float32)

    out_ref[...] = jnp.concatenate([act(ga), act(gb)], axis=-1)


def _dense(xij, edge_attr, w_cat, b_cat, e_off, ec):
    eh = ec // 2
    nblk = eh // BH
    ea_a = e_off // BH
    ea_b = (e_off + eh) // BH
    return pl.pallas_call(
        _dense_body,
        grid=(nblk,),
        in_specs=[
            pl.BlockSpec((BH, 2 * C), lambda i: (i, 0)),
            pl.BlockSpec((BH, 2 * C), lambda i, n=nblk: (i + n, 0)),
            pl.BlockSpec((BH, ED), lambda i, o=ea_a: (i + o, 0)),
            pl.BlockSpec((BH, ED), lambda i, o=ea_b: (i + o, 0)),
            pl.BlockSpec((Z, 2 * C), lambda i: (0, 0)),
            pl.BlockSpec((1, 2 * C), lambda i: (0, 0)),
        ],
        out_specs=pl.BlockSpec((BH, 2 * C), lambda i: (i, 0)),
        out_shape=jax.ShapeDtypeStruct((eh, 2 * C), jnp.float32),
    )(xij, xij, edge_attr, edge_attr, w_cat, b_cat)


BN = 2000


def _combine_body(x_ref, *refs):
    p_refs, out_ref = refs[:-1], refs[-1]
    s = x_ref[...]
    for p in p_refs:
        s = s + p[:, :C] + p[:, C:]
    out_ref[...] = jnp.maximum(s, 0.0)


def _combine(x, partials):
    return pl.pallas_call(
        _combine_body,
        grid=(N // BN,),
        in_specs=[pl.BlockSpec((BN, C), lambda i: (i, 0))]
        + [pl.BlockSpec((BN, 2 * C), lambda i: (i, 0)) for _ in partials],
        out_specs=pl.BlockSpec((BN, C), lambda i: (i, 0)),
        out_shape=jax.ShapeDtypeStruct((N, C), jnp.float32),
    )(x, *partials)


def kernel(x, edge_index, edge_attr, W_f, b_f, W_s, b_s):
    gathers, scatters = _sc_kernels()
    ei = edge_index if edge_index.dtype == jnp.int32 else edge_index.astype(jnp.int32)
    w_cat = jnp.concatenate([W_f, W_s], axis=1)
    b_cat = jnp.concatenate([b_f, b_s]).reshape(1, 2 * C)
    prev = jnp.zeros((N, 2 * C), jnp.float32)
    for (e_off, ec), g, s in zip(CHAINS, gathers, scatters):
        xij = g(x, ei)
        msg2 = _dense(xij, edge_attr, w_cat, b_cat, e_off, ec)
        prev = s(msg2, ei, prev)
    return _combine(x, [prev])


# speedup vs baseline: 1.1372x; 1.0351x over previous
import functools

import jax
import jax.numpy as jnp
from jax import lax
from jax.experimental import pallas as pl
from jax.experimental.pallas import tpu as pltpu
from jax.experimental.pallas import tpu_sc as plsc

N = 10000
E = 320000
C = 64
ED = 128
Z = 2 * C + ED

NC = 2
NS = 16
NW = NC * NS
RPT = N // NS

CHAINS = ((0, 102400), (102400, 140800), (243200, 76800))

G_CH = 40
G_K = 5
G_WV = G_K * G_CH
G_R = 2

S_CH = 40
S_K = 5
S_WV = S_K * S_CH
S_R = 2


def _sc_gather_body(ept, nwave, e_off, x_hbm, ei_hbm, xij_hbm,
                    tbl, dst_v, src_v, bi, bj, sem_g, sem_w):
    cid = lax.axis_index("c")
    sid = lax.axis_index("s")
    wid = sid * NC + cid
    base = wid * ept
    rows = sid * RPT
    pltpu.sync_copy(x_hbm.at[pl.ds(rows, RPT)], tbl.at[pl.ds(rows, RPT)])
    pltpu.sync_copy(ei_hbm.at[1, pl.ds(e_off + base, ept)], dst_v)
    pltpu.sync_copy(ei_hbm.at[0, pl.ds(e_off + base, ept)], src_v)
    plsc.subcore_barrier()

    def wave(w, carry):
        s = w % G_R

        @pl.when(w >= G_R)
        def _drain():
            pltpu.make_async_copy(
                bi.at[s], xij_hbm.at[pl.ds(base, G_WV), pl.ds(0, C)], sem_w).wait()
            pltpu.make_async_copy(
                bj.at[s], xij_hbm.at[pl.ds(base, G_WV), pl.ds(C, C)], sem_w).wait()

        cps = []
        for b in range(G_K):
            o = w * G_WV + b * G_CH
            cps.append(pltpu.async_copy(
                tbl.at[dst_v.at[pl.ds(o, G_CH)]],
                bi.at[s, pl.ds(b * G_CH, G_CH)], sem_g))
            cps.append(pltpu.async_copy(
                tbl.at[src_v.at[pl.ds(o, G_CH)]],
                bj.at[s, pl.ds(b * G_CH, G_CH)], sem_g))
        for cp in cps:
            cp.wait()
        o = base + w * G_WV
        pltpu.async_copy(bi.at[s], xij_hbm.at[pl.ds(o, G_WV), pl.ds(0, C)], sem_w)
        pltpu.async_copy(bj.at[s], xij_hbm.at[pl.ds(o, G_WV), pl.ds(C, C)], sem_w)
        return carry

    lax.fori_loop(0, nwave, wave, 0)
    for _ in range(G_R):
        pltpu.make_async_copy(
            bi.at[0], xij_hbm.at[pl.ds(base, G_WV), pl.ds(0, C)], sem_w).wait()
        pltpu.make_async_copy(
            bj.at[0], xij_hbm.at[pl.ds(base, G_WV), pl.ds(C, C)], sem_w).wait()


def _sc_scatter_body(rpt, nwave, e_off, eh, msg_hbm, ei_hbm, prev_hbm, out_hbm,
                     mba, mbb, iba, ibb, acc, sem_l, sem_s):
    cid = lax.axis_index("c")
    sid = lax.axis_index("s")
    wid = sid * NC + cid
    base = wid * rpt
    rows = sid * RPT
    pltpu.sync_copy(prev_hbm.at[pl.ds(rows, RPT), pl.ds(cid * C, C)],
                    acc.at[pl.ds(rows, RPT)])
    plsc.subcore_barrier()

    def wave(w, carry):
        s = w % S_R

        @pl.when(w >= S_R)
        def _drain():
            for b in range(S_K):
                pltpu.make_async_copy(
                    mba.at[s, b], acc.at[pl.ds(0, S_CH)], sem_s).wait()
                pltpu.make_async_copy(
                    mbb.at[s, b], acc.at[pl.ds(0, S_CH)], sem_s).wait()

        loads = []
        for b in range(S_K):
            o = base + w * S_WV + b * S_CH
            loads.append(pltpu.async_copy(
                ei_hbm.at[1, pl.ds(e_off + o, S_CH)], iba.at[s, b], sem_l))
            loads.append(pltpu.async_copy(
                ei_hbm.at[1, pl.ds(e_off + eh + o, S_CH)], ibb.at[s, b], sem_l))
            loads.append(pltpu.async_copy(
                msg_hbm.at[pl.ds(o, S_CH), pl.ds(0, C)], mba.at[s, b], sem_l))
            loads.append(pltpu.async_copy(
                msg_hbm.at[pl.ds(o, S_CH), pl.ds(C, C)], mbb.at[s, b], sem_l))
        for cp in loads:
            cp.wait()
        for b in range(S_K):
            pltpu.async_copy(mba.at[s, b], acc.at[iba.at[s, b]], sem_s, add=True)
            pltpu.async_copy(mbb.at[s, b], acc.at[ibb.at[s, b]], sem_s, add=True)
        return carry

    lax.fori_loop(0, nwave, wave, 0)
    for _ in range(S_R):
        for b in range(S_K):
            pltpu.make_async_copy(
                mba.at[0, b], acc.at[pl.ds(0, S_CH)], sem_s).wait()
            pltpu.make_async_copy(
                mbb.at[0, b], acc.at[pl.ds(0, S_CH)], sem_s).wait()
    plsc.subcore_barrier()
    pltpu.sync_copy(acc.at[pl.ds(rows, RPT)],
                    out_hbm.at[pl.ds(rows, RPT), pl.ds(cid * C, C)])


@functools.cache
def _sc_kernels():
    mesh = plsc.VectorSubcoreMesh(core_axis_name="c", subcore_axis_name="s",
                                  num_cores=NC, num_subcores=NS)
    params = pltpu.CompilerParams(use_tc_tiling_on_sc=False)
    gathers, scatters = [], []
    for e_off, ec in CHAINS:
        ept = ec // NW
        gathers.append(pl.kernel(
            functools.partial(_sc_gather_body, ept, ept // G_WV, e_off),
            out_type=jax.ShapeDtypeStruct((ec, 2 * C), jnp.float32),
            mesh=mesh,
            compiler_params=params,
            scratch_types=[
                pltpu.VMEM_SHARED((N, C), jnp.float32),
                pltpu.VMEM((ept,), jnp.int32),
                pltpu.VMEM((ept,), jnp.int32),
                pltpu.VMEM((G_R, G_WV, C), jnp.float32),
                pltpu.VMEM((G_R, G_WV, C), jnp.float32),
                pltpu.SemaphoreType.DMA,
                pltpu.SemaphoreType.DMA,
            ],
        ))
        rpt = (ec // 2) // NW
        scatters.append(pl.kernel(
            functools.partial(_sc_scatter_body, rpt, rpt // S_WV, e_off, ec // 2),
            out_type=jax.ShapeDtypeStruct((N, 2 * C), jnp.float32),
            mesh=mesh,
            compiler_params=params,
            scratch_types=[
                pltpu.VMEM((S_R, S_K, S_CH, C), jnp.float32),
                pltpu.VMEM((S_R, S_K, S_CH, C), jnp.float32),
                pltpu.VMEM((S_R, S_K, S_CH), jnp.int32),
                pltpu.VMEM((S_R, S_K, S_CH), jnp.int32),
                pltpu.VMEM_SHARED((N, C), jnp.float32),
                pltpu.SemaphoreType.DMA,
                pltpu.SemaphoreType.DMA,
            ],
        ))
    return gathers, scatters


BH = 3200


def _dense_body(xa_ref, xb_ref, ea_ref, eb_ref, w_ref, b_ref, out_ref):
    za = jnp.concatenate([xa_ref[...], ea_ref[...]], axis=-1)
    zb = jnp.concatenate([xb_ref[...], eb_ref[...]], axis=-1)
    ga = jnp.dot(za, w_ref[...], preferred_element_type=jnp.float32) + b_ref[...]
    gb = jnp.dot(zb, w_ref[...], preferred_element_type=jnp.float32) + b_ref[...]

    def act(gs):
        h = gs.astype(jnp.bfloat16)
        g = h[:, :C]
        s = h[:, C:]
        gate = jnp.bfloat16(0.5) + jnp.bfloat16(0.5) * jnp.tanh(jnp.bfloat16(0.5) * g)
        core = (jnp.maximum(s, jnp.bfloat16(0.0))
                + jnp.log1p(jnp.exp(-jnp.abs(s))))
        return (gate * core).astype(jnp.float32)

    out_ref[...] = jnp.concatenate([act(ga), act(gb)], axis=-1)


def _dense(xij, edge_attr, w_cat, b_cat, e_off, ec):
    eh = ec // 2
    nblk = eh // BH
    ea_a = e_off // BH
    ea_b = (e_off + eh) // BH
    return pl.pallas_call(
        _dense_body,
        grid=(nblk,),
        in_specs=[
            pl.BlockSpec((BH, 2 * C), lambda i: (i, 0)),
            pl.BlockSpec((BH, 2 * C), lambda i, n=nblk: (i + n, 0)),
            pl.BlockSpec((BH, ED), lambda i, o=ea_a: (i + o, 0)),
            pl.BlockSpec((BH, ED), lambda i, o=ea_b: (i + o, 0)),
            pl.BlockSpec((Z, 2 * C), lambda i: (0, 0)),
            pl.BlockSpec((1, 2 * C), lambda i: (0, 0)),
        ],
        out_specs=pl.BlockSpec((BH, 2 * C), lambda i: (i, 0)),
        out_shape=jax.ShapeDtypeStruct((eh, 2 * C), jnp.float32),
    )(xij, xij, edge_attr, edge_attr, w_cat, b_cat)


BN = 2000


def _combine_body(x_ref, *refs):
    p_refs, out_ref = refs[:-1], refs[-1]
    s = x_ref[...]
    for p in p_refs:
        s = s + p[:, :C] + p[:, C:]
    out_ref[...] = jnp.maximum(s, 0.0)


def _combine(x, partials):
    return pl.pallas_call(
        _combine_body,
        grid=(N // BN,),
        in_specs=[pl.BlockSpec((BN, C), lambda i: (i, 0))]
        + [pl.BlockSpec((BN, 2 * C), lambda i: (i, 0)) for _ in partials],
        out_specs=pl.BlockSpec((BN, C), lambda i: (i, 0)),
        out_shape=jax.ShapeDtypeStruct((N, C), jnp.float32),
    )(x, *partials)


def kernel(x, edge_index, edge_attr, W_f, b_f, W_s, b_s):
    gathers, scatters = _sc_kernels()
    ei = edge_index if edge_index.dtype == jnp.int32 else edge_index.astype(jnp.int32)
    w_cat = jnp.concatenate([W_f, W_s], axis=1)
    b_cat = jnp.concatenate([b_f, b_s]).reshape(1, 2 * C)
    prev = jnp.zeros((N, 2 * C), jnp.float32)
    for (e_off, ec), g, s in zip(CHAINS, gathers, scatters):
        xij = g(x, ei)
        msg2 = _dense(xij, edge_attr, w_cat, b_cat, e_off, ec)
        prev = s(msg2, ei, prev)
    return _combine(x, [prev])
